# baseline (speedup 1.0000x reference)
import functools

import jax
import jax.numpy as jnp
from jax.experimental import pallas as pl
from jax.experimental.pallas import tpu as pltpu

K = 1024
D = 64
BETA = 0.25
N = 16 * 32 * 32
BN = 1024
NSTEPS = N // BN


def _vq_body(f_ref, a_ref, b_ref, emb_ref, enc_ref, q_ref, loss_ref, perp_ref,
             cnt_ref, sse_ref):
    i = pl.program_id(0)

    @pl.when(i == 0)
    def _init():
        cnt_ref[...] = jnp.zeros_like(cnt_ref)
        sse_ref[0] = 0.0

    f = f_ref[...]
    emb = emb_ref[...]
    m2 = 2.0 * jax.lax.dot_general(
        f, emb, (((1,), (1,)), ((), ())),
        preferred_element_type=jnp.float32)
    dist = (a_ref[...] + b_ref[...]) - m2

    iota = jax.lax.broadcasted_iota(jnp.int32, (BN, K), 1)
    mn = jnp.min(dist, axis=1, keepdims=True)
    idx = jnp.min(jnp.where(dist == mn, iota, K), axis=1, keepdims=True)
    enc = (iota == idx).astype(jnp.float32)

    q = jax.lax.dot_general(
        enc, emb, (((1,), (0,)), ((), ())),
        preferred_element_type=jnp.float32)
    diff = q - f
    sse_ref[0] += jnp.sum(diff * diff)
    cnt_ref[...] += jnp.sum(enc, axis=0, keepdims=True)
    q_ref[...] = f + (q - f)

    @pl.when(i == NSTEPS - 1)
    def _fin():
        mse = sse_ref[0] / jnp.float32(N * D)
        loss_ref[...] = jnp.full((1, 1), mse + jnp.float32(BETA) * mse,
                                 dtype=jnp.float32)
        p = cnt_ref[...] * jnp.float32(1.0 / N)
        plogp = p * jnp.log(p + jnp.float32(1e-10))
        perp_ref[...] = jnp.exp(-jnp.sum(plogp, axis=1, keepdims=True))


@functools.partial(jax.jit, static_argnames=())
def kernel(x, emb_weight):
    xp = jnp.transpose(x, (0, 2, 3, 1))
    latents_shape = xp.shape
    flat = xp.reshape(-1, D)
    a = jnp.sum(flat ** 2, axis=1, keepdims=True)
    b = jnp.sum(emb_weight ** 2, axis=1).reshape(1, K)

    grid = (NSTEPS,)
    enc, qst, loss, perp = pl.pallas_call(
        _vq_body,
        grid=grid,
        in_specs=[
            pl.BlockSpec((BN, D), lambda i: (i, 0)),
            pl.BlockSpec((BN, 1), lambda i: (i, 0)),
            pl.BlockSpec((1, K), lambda i: (0, 0)),
            pl.BlockSpec((K, D), lambda i: (0, 0)),
        ],
        out_specs=[
            pl.BlockSpec((BN, K), lambda i: (i, 0)),
            pl.BlockSpec((BN, D), lambda i: (i, 0)),
            pl.BlockSpec((1, 1), lambda i: (0, 0)),
            pl.BlockSpec((1, 1), lambda i: (0, 0)),
        ],
        out_shape=[
            jax.ShapeDtypeStruct((N, K), jnp.float32),
            jax.ShapeDtypeStruct((N, D), jnp.float32),
            jax.ShapeDtypeStruct((1, 1), jnp.float32),
            jax.ShapeDtypeStruct((1, 1), jnp.float32),
        ],
        scratch_shapes=[
            pltpu.VMEM((1, K), jnp.float32),
            pltpu.SMEM((1,), jnp.float32),
        ],
    )(flat, a, b, emb_weight)

    quantized = jnp.transpose(qst.reshape(latents_shape), (0, 3, 1, 2))
    return (quantized, loss[0, 0], perp[0, 0], enc)


# speedup vs baseline: 1.0093x; 1.0093x over previous
import functools

import jax
import jax.numpy as jnp
from jax.experimental import pallas as pl
from jax.experimental.pallas import tpu as pltpu

K = 1024
D = 64
BETA = 0.25
N = 16 * 32 * 32
BN = 1024
NSTEPS = N // BN


def _vq_body(f_ref, a_ref, b_ref, emb_ref, enc_ref, q_ref, loss_ref, perp_ref,
             cnt_ref, sse_ref):
    i = pl.program_id(0)

    @pl.when(i == 0)
    def _init():
        cnt_ref[...] = jnp.zeros_like(cnt_ref)
        sse_ref[0] = 0.0

    f = f_ref[...]
    emb = emb_ref[...]
    m2 = 2.0 * jax.lax.dot_general(
        f, emb, (((1,), (1,)), ((), ())),
        preferred_element_type=jnp.float32)
    dist = (a_ref[...] + b_ref[...]) - m2

    iota = jax.lax.broadcasted_iota(jnp.int32, (BN, K), 1)
    mn = jnp.min(dist, axis=1, keepdims=True)
    idx = jnp.min(jnp.where(dist == mn, iota, K), axis=1, keepdims=True)
    enc = (iota == idx).astype(jnp.float32)
    enc_ref[...] = enc

    q = jax.lax.dot_general(
        enc, emb, (((1,), (0,)), ((), ())),
        preferred_element_type=jnp.float32)
    diff = q - f
    sse_ref[0] += jnp.sum(diff * diff)
    cnt_ref[...] += jnp.sum(enc, axis=0, keepdims=True)
    q_ref[...] = f + (q - f)

    @pl.when(i == NSTEPS - 1)
    def _fin():
        mse = sse_ref[0] / jnp.float32(N * D)
        loss_ref[...] = jnp.full((1, 1), mse + jnp.float32(BETA) * mse,
                                 dtype=jnp.float32)
        p = cnt_ref[...] * jnp.float32(1.0 / N)
        plogp = p * jnp.log(p + jnp.float32(1e-10))
        perp_ref[...] = jnp.exp(-jnp.sum(plogp, axis=1, keepdims=True))


@functools.partial(jax.jit, static_argnames=())
def kernel(x, emb_weight):
    xp = jnp.transpose(x, (0, 2, 3, 1))
    latents_shape = xp.shape
    flat = xp.reshape(-1, D)
    a = jnp.sum(flat ** 2, axis=1, keepdims=True)
    b = jnp.sum(emb_weight ** 2, axis=1).reshape(1, K)

    grid = (NSTEPS,)
    enc, qst, loss, perp = pl.pallas_call(
        _vq_body,
        grid=grid,
        in_specs=[
            pl.BlockSpec((BN, D), lambda i: (i, 0)),
            pl.BlockSpec((BN, 1), lambda i: (i, 0)),
            pl.BlockSpec((1, K), lambda i: (0, 0)),
            pl.BlockSpec((K, D), lambda i: (0, 0)),
        ],
        out_specs=[
            pl.BlockSpec((BN, K), lambda i: (i, 0)),
            pl.BlockSpec((BN, D), lambda i: (i, 0)),
            pl.BlockSpec((1, 1), lambda i: (0, 0)),
            pl.BlockSpec((1, 1), lambda i: (0, 0)),
        ],
        out_shape=[
            jax.ShapeDtypeStruct((N, K), jnp.float32),
            jax.ShapeDtypeStruct((N, D), jnp.float32),
            jax.ShapeDtypeStruct((1, 1), jnp.float32),
            jax.ShapeDtypeStruct((1, 1), jnp.float32),
        ],
        scratch_shapes=[
            pltpu.VMEM((1, K), jnp.float32),
            pltpu.SMEM((1,), jnp.float32),
        ],
    )(flat, a, b, emb_weight)

    quantized = jnp.transpose(qst.reshape(latents_shape), (0, 3, 1, 2))
    return (quantized, loss[0, 0], perp[0, 0], enc)
